# baseline (speedup 1.0000x reference)
import functools

import jax
import jax.numpy as jnp
from jax import lax
from jax.experimental import pallas as pl
from jax.experimental.pallas import tpu as pltpu
from jax.experimental.pallas import tpu_sc as plsc

NUM_LODS = 10
TEX_H = 512
TEX_W = 512
NUM_CHANNELS = 11

NC = 2
NS = 16
NW = NC * NS
L = 16

CHUNK = 2048
SUB = 128
NSUB = CHUNK // SUB
DPAD = 16


def _compute_indices(slab_v, idx_v, iota):
    def jloop(j, c):
        for l in range(4):
            off = j * (4 * L) + l * L
            p = (off + iota) * 3
            ys = plsc.load_gather(slab_v, [p])
            xs = plsc.load_gather(slab_v, [p + 1])
            lods = plsc.load_gather(slab_v, [p + 2])
            sy = lax.shift_right_logical(ys, lods)
            sx = lax.shift_right_logical(xs, lods)
            t = (1 << 20) - lax.shift_right_logical(
                jnp.full((L,), 1 << 20, jnp.int32), 2 * lods
            )
            base_row = t * jnp.int32(-1431655765)
            idx = base_row + lax.shift_left(sy, 9 - lods) + sx
            idx_v[j // 2, pl.ds((j % 2) * (4 * L) + l * L, L)] = idx
        return c

    lax.fori_loop(0, CHUNK // (4 * L), jloop, 0)


def _transpose_rows(rows_v, tr_v, iota):
    def jloop(j, c):
        s = j * L + iota
        for ch in range(DPAD):
            w = plsc.load_gather(rows_v, [s, jnp.full((L,), ch, jnp.int32)])
            tr_v[ch, pl.ds(j * L, L)] = w
        return c

    lax.fori_loop(0, CHUNK // L, jloop, 0)


def _tex_kernel_body(
    table_hbm, bi_hbm, out_hbm,
    slab0, slab1, idx0, idx1, rows0, rows1, tr0,
    sa0, sa1, sc0, sc1, sd0, sd1,
):
    wid = lax.axis_index("s") * NC + lax.axis_index("c")
    batch = out_hbm.shape[1]
    bpw = batch // NW
    nchunk = bpw // CHUNK
    iota = lax.iota(jnp.int32, L)

    slabs = [slab0, slab1]
    idxs = [idx0, idx1]
    rows = [rows0, rows1]
    trs = [tr0, tr0]
    sas = [sa0, sa1]
    scs = [sc0, sc1]
    sds = [sd0, sd1]

    def fire_a(k):
        base = wid * bpw + k * CHUNK
        return pltpu.async_copy(
            bi_hbm.at[pl.ds(base * 3, CHUNK * 3)], slabs[k % 2], sas[k % 2]
        )

    def fire_c(k):
        b = k % 2

        def go(j, c):
            pltpu.async_copy(
                table_hbm.at[idxs[b].at[j]],
                rows[b].at[pl.ds(j * SUB, SUB)],
                scs[b],
            )
            return c

        lax.fori_loop(0, NSUB, go, 0)

    def wait_c(k):
        b = k % 2

        def wt(j, c):
            pltpu.make_async_copy(
                table_hbm.at[idxs[b].at[0]],
                rows[b].at[pl.ds(0, SUB)],
                scs[b],
            ).wait()
            return c

        lax.fori_loop(0, NSUB, wt, 0)

    def fire_d(k):
        base = wid * bpw + k * CHUNK
        cps = [
            pltpu.async_copy(
                trs[k % 2].at[ch], out_hbm.at[ch, pl.ds(base, CHUNK)], sds[k % 2]
            )
            for ch in range(DPAD)
        ]
        return cps

    a_pend = {0: fire_a(0), 1: fire_a(1)}
    c_live = set()
    d_pend = {}

    def finish(k):
        c_live.discard(k)
        wait_c(k)
        if k - 1 in d_pend:
            for dp in d_pend.pop(k - 1):
                dp.wait()
        _transpose_rows(rows[k % 2], trs[k % 2], iota)
        d_pend[k] = fire_d(k)

    for k in range(nchunk):
        a_pend.pop(k).wait()
        _compute_indices(slabs[k % 2], idxs[k % 2], iota)
        if k + 2 < nchunk:
            a_pend[k + 2] = fire_a(k + 2)
        if k - 1 in c_live:
            finish(k - 1)
        fire_c(k)
        c_live.add(k)
    finish(nchunk - 1)
    for k in sorted(d_pend):
        for dp in d_pend.pop(k):
            dp.wait()


def _make_tex_gather(batch):
    mesh = plsc.VectorSubcoreMesh(
        core_axis_name="c", subcore_axis_name="s", num_cores=NC, num_subcores=NS
    )
    return functools.partial(
        pl.kernel,
        out_type=jax.ShapeDtypeStruct((DPAD, batch), jnp.float32),
        mesh=mesh,
        scratch_types=[
            pltpu.VMEM((CHUNK * 3,), jnp.int32),
            pltpu.VMEM((CHUNK * 3,), jnp.int32),
            pltpu.VMEM((NSUB, SUB), jnp.int32),
            pltpu.VMEM((NSUB, SUB), jnp.int32),
            pltpu.VMEM((CHUNK, DPAD), jnp.float32),
            pltpu.VMEM((CHUNK, DPAD), jnp.float32),
            pltpu.VMEM((DPAD, CHUNK), jnp.float32),
            pltpu.SemaphoreType.DMA,
            pltpu.SemaphoreType.DMA,
            pltpu.SemaphoreType.DMA,
            pltpu.SemaphoreType.DMA,
            pltpu.SemaphoreType.DMA,
            pltpu.SemaphoreType.DMA,
        ],
        compiler_params=pltpu.CompilerParams(
            needs_layout_passes=False, use_tc_tiling_on_sc=False
        ),
    )(_tex_kernel_body)


def kernel(lod_cache, batch_index):
    batch = batch_index.shape[0]
    parts = [
        lax.slice(
            lod_cache,
            (l, 0, 0, 0),
            (l + 1, TEX_H >> l, TEX_W >> l, NUM_CHANNELS),
        ).reshape(-1, NUM_CHANNELS)
        for l in range(NUM_LODS)
    ]
    table = jnp.concatenate(parts, axis=0)
    nrows = table.shape[0]
    rpad = (-nrows) % 8
    table = jnp.pad(table, ((0, rpad), (0, DPAD - NUM_CHANNELS)))
    bi = batch_index.astype(jnp.int32).reshape(-1)
    out = _make_tex_gather(batch)(table, bi)
    return out[:NUM_CHANNELS, :].T


# speedup vs baseline: 1.3296x; 1.3296x over previous
import functools

import jax
import jax.numpy as jnp
from jax import lax
from jax.experimental import pallas as pl
from jax.experimental.pallas import tpu as pltpu
from jax.experimental.pallas import tpu_sc as plsc

NUM_LODS = 10
TEX_H = 512
TEX_W = 512
NUM_CHANNELS = 11

NC = 2
NS = 16
NW = NC * NS
L = 16

CHUNK = 2048
SUB = 128
NSUB = CHUNK // SUB
DPAD = 16


def _compute_indices(slab_v, idx_v, iota):
    def jloop(j, c):
        for l in range(4):
            off = j * (4 * L) + l * L
            p = (off + iota) * 3
            ys = plsc.load_gather(slab_v, [p])
            xs = plsc.load_gather(slab_v, [p + 1])
            lods = plsc.load_gather(slab_v, [p + 2])
            sy = lax.shift_right_logical(ys, lods)
            sx = lax.shift_right_logical(xs, lods)
            t = (1 << 20) - lax.shift_right_logical(
                jnp.full((L,), 1 << 20, jnp.int32), 2 * lods
            )
            base_row = t * jnp.int32(-1431655765)
            idx = base_row + lax.shift_left(sy, 9 - lods) + sx
            idx_v[j // 2, pl.ds((j % 2) * (4 * L) + l * L, L)] = idx
        return c

    lax.fori_loop(0, CHUNK // (4 * L), jloop, 0)


def _tex_kernel_body(
    table_hbm, bi_hbm, out_hbm,
    slab0, slab1, idx0, idx1, rows0, rows1,
    sa0, sa1, sc0, sc1, sd0, sd1,
):
    wid = lax.axis_index("s") * NC + lax.axis_index("c")
    batch = out_hbm.shape[0] * SUB
    bpw = batch // NW
    nchunk = bpw // CHUNK
    iota = lax.iota(jnp.int32, L)

    slabs = [slab0, slab1]
    idxs = [idx0, idx1]
    rows = [rows0, rows1]
    sas = [sa0, sa1]
    scs = [sc0, sc1]
    sds = [sd0, sd1]

    def fire_a(k):
        base = wid * bpw + k * CHUNK
        return pltpu.async_copy(
            bi_hbm.at[pl.ds(base * 3, CHUNK * 3)], slabs[k % 2], sas[k % 2]
        )

    def fire_c(k):
        b = k % 2

        def go(j, c):
            pltpu.async_copy(
                table_hbm.at[idxs[b].at[j]],
                rows[b].at[j],
                scs[b],
            )
            return c

        lax.fori_loop(0, NSUB, go, 0)

    def wait_c(k):
        b = k % 2

        def wt(j, c):
            pltpu.make_async_copy(
                table_hbm.at[idxs[b].at[0]],
                rows[b].at[0],
                scs[b],
            ).wait()
            return c

        lax.fori_loop(0, NSUB, wt, 0)

    def fire_d(k):
        base = wid * bpw + k * CHUNK
        return [
            pltpu.async_copy(
                rows[k % 2],
                out_hbm.at[pl.ds(base // SUB, CHUNK // SUB)],
                sds[k % 2],
            )
        ]

    a_pend = {0: fire_a(0), 1: fire_a(1)}
    c_live = set()
    d_pend = {}

    def finish(k):
        c_live.discard(k)
        wait_c(k)
        if k - 2 in d_pend:
            for dp in d_pend.pop(k - 2):
                dp.wait()
        d_pend[k] = fire_d(k)

    for k in range(nchunk):
        a_pend.pop(k).wait()
        _compute_indices(slabs[k % 2], idxs[k % 2], iota)
        if k + 2 < nchunk:
            a_pend[k + 2] = fire_a(k + 2)
        if k - 1 in c_live:
            finish(k - 1)
        fire_c(k)
        c_live.add(k)
    finish(nchunk - 1)
    for k in sorted(d_pend):
        for dp in d_pend.pop(k):
            dp.wait()


def _make_tex_gather(batch):
    mesh = plsc.VectorSubcoreMesh(
        core_axis_name="c", subcore_axis_name="s", num_cores=NC, num_subcores=NS
    )
    return functools.partial(
        pl.kernel,
        out_type=jax.ShapeDtypeStruct((batch // SUB, SUB, DPAD), jnp.float32),
        mesh=mesh,
        scratch_types=[
            pltpu.VMEM((CHUNK * 3,), jnp.int32),
            pltpu.VMEM((CHUNK * 3,), jnp.int32),
            pltpu.VMEM((NSUB, SUB), jnp.int32),
            pltpu.VMEM((NSUB, SUB), jnp.int32),
            pltpu.VMEM((NSUB, SUB, DPAD), jnp.float32),
            pltpu.VMEM((NSUB, SUB, DPAD), jnp.float32),
            pltpu.SemaphoreType.DMA,
            pltpu.SemaphoreType.DMA,
            pltpu.SemaphoreType.DMA,
            pltpu.SemaphoreType.DMA,
            pltpu.SemaphoreType.DMA,
            pltpu.SemaphoreType.DMA,
        ],
        compiler_params=pltpu.CompilerParams(
            needs_layout_passes=False, use_tc_tiling_on_sc=False
        ),
    )(_tex_kernel_body)


def kernel(lod_cache, batch_index):
    batch = batch_index.shape[0]
    parts = [
        lax.slice(
            lod_cache,
            (l, 0, 0, 0),
            (l + 1, TEX_H >> l, TEX_W >> l, NUM_CHANNELS),
        ).reshape(-1, NUM_CHANNELS)
        for l in range(NUM_LODS)
    ]
    table = jnp.concatenate(parts, axis=0)
    nrows = table.shape[0]
    rpad = (-nrows) % 8
    table = jnp.pad(table, ((0, rpad), (0, DPAD - NUM_CHANNELS)))
    bi = batch_index.astype(jnp.int32).reshape(-1)
    out = _make_tex_gather(batch)(table, bi)
    return out.reshape(batch, DPAD)[:, :NUM_CHANNELS]


# speedup vs baseline: 1.8598x; 1.3987x over previous
import functools

import jax
import jax.numpy as jnp
from jax import lax
from jax.experimental import pallas as pl
from jax.experimental.pallas import tpu as pltpu
from jax.experimental.pallas import tpu_sc as plsc

NUM_LODS = 10
TEX_H = 512
TEX_W = 512
NUM_CHANNELS = 11

NC = 2
NS = 16
NW = NC * NS
L = 16

CHUNK = 2048
SUB = 128
NSUB = CHUNK // SUB
DPAD = 16


def _compute_indices(slab_v, idx_v, iota):
    def jloop(j, c):
        for l in range(4):
            off = j * (4 * L) + l * L
            ys = slab_v[pl.ds(off, L)]
            xs = slab_v[pl.ds(CHUNK + off, L)]
            lods = slab_v[pl.ds(2 * CHUNK + off, L)]
            sy = lax.shift_right_logical(ys, lods)
            sx = lax.shift_right_logical(xs, lods)
            t = (1 << 20) - lax.shift_right_logical(
                jnp.full((L,), 1 << 20, jnp.int32), 2 * lods
            )
            base_row = t * jnp.int32(-1431655765)
            idx = base_row + lax.shift_left(sy, 9 - lods) + sx
            idx_v[j // 2, pl.ds((j % 2) * (4 * L) + l * L, L)] = idx
        return c

    lax.fori_loop(0, CHUNK // (4 * L), jloop, 0)


def _tex_kernel_body(
    table_hbm, bi_hbm, out_hbm,
    slab0, slab1, idx0, idx1, rows0, rows1,
    sa0, sa1, sc0, sc1, sd0, sd1,
):
    wid = lax.axis_index("s") * NC + lax.axis_index("c")
    batch = out_hbm.shape[0] * SUB
    bpw = batch // NW
    nchunk = bpw // CHUNK
    iota = lax.iota(jnp.int32, L)

    slabs = [slab0, slab1]
    idxs = [idx0, idx1]
    rows = [rows0, rows1]
    sas = [sa0, sa1]
    scs = [sc0, sc1]
    sds = [sd0, sd1]

    def fire_a(k):
        base = wid * bpw + k * CHUNK
        return [
            pltpu.async_copy(
                bi_hbm.at[pl.ds(p * batch + base, CHUNK)],
                slabs[k % 2].at[pl.ds(p * CHUNK, CHUNK)],
                sas[k % 2],
            )
            for p in range(3)
        ]

    def fire_c(k):
        b = k % 2

        def go(j, c):
            pltpu.async_copy(
                table_hbm.at[idxs[b].at[j]],
                rows[b].at[j],
                scs[b],
            )
            return c

        lax.fori_loop(0, NSUB, go, 0)

    def wait_c(k):
        b = k % 2

        def wt(j, c):
            pltpu.make_async_copy(
                table_hbm.at[idxs[b].at[0]],
                rows[b].at[0],
                scs[b],
            ).wait()
            return c

        lax.fori_loop(0, NSUB, wt, 0)

    def fire_d(k):
        base = wid * bpw + k * CHUNK
        return [
            pltpu.async_copy(
                rows[k % 2],
                out_hbm.at[pl.ds(base // SUB, CHUNK // SUB)],
                sds[k % 2],
            )
        ]

    a_pend = {0: fire_a(0), 1: fire_a(1)}
    c_live = set()
    d_pend = {}

    def finish(k):
        c_live.discard(k)
        wait_c(k)
        if k - 2 in d_pend:
            for dp in d_pend.pop(k - 2):
                dp.wait()
        d_pend[k] = fire_d(k)

    for k in range(nchunk):
        for ap in a_pend.pop(k):
            ap.wait()
        _compute_indices(slabs[k % 2], idxs[k % 2], iota)
        if k + 2 < nchunk:
            a_pend[k + 2] = fire_a(k + 2)
        if k - 1 in c_live:
            finish(k - 1)
        fire_c(k)
        c_live.add(k)
    finish(nchunk - 1)
    for k in sorted(d_pend):
        for dp in d_pend.pop(k):
            dp.wait()


def _make_tex_gather(batch):
    mesh = plsc.VectorSubcoreMesh(
        core_axis_name="c", subcore_axis_name="s", num_cores=NC, num_subcores=NS
    )
    return functools.partial(
        pl.kernel,
        out_type=jax.ShapeDtypeStruct((batch // SUB, SUB, DPAD), jnp.float32),
        mesh=mesh,
        scratch_types=[
            pltpu.VMEM((CHUNK * 3,), jnp.int32),
            pltpu.VMEM((CHUNK * 3,), jnp.int32),
            pltpu.VMEM((NSUB, SUB), jnp.int32),
            pltpu.VMEM((NSUB, SUB), jnp.int32),
            pltpu.VMEM((NSUB, SUB, DPAD), jnp.float32),
            pltpu.VMEM((NSUB, SUB, DPAD), jnp.float32),
            pltpu.SemaphoreType.DMA,
            pltpu.SemaphoreType.DMA,
            pltpu.SemaphoreType.DMA,
            pltpu.SemaphoreType.DMA,
            pltpu.SemaphoreType.DMA,
            pltpu.SemaphoreType.DMA,
        ],
        compiler_params=pltpu.CompilerParams(
            needs_layout_passes=False, use_tc_tiling_on_sc=False
        ),
    )(_tex_kernel_body)


def kernel(lod_cache, batch_index):
    batch = batch_index.shape[0]
    parts = [
        lax.slice(
            lod_cache,
            (l, 0, 0, 0),
            (l + 1, TEX_H >> l, TEX_W >> l, NUM_CHANNELS),
        ).reshape(-1, NUM_CHANNELS)
        for l in range(NUM_LODS)
    ]
    table = jnp.concatenate(parts, axis=0)
    nrows = table.shape[0]
    rpad = (-nrows) % 8
    table = jnp.pad(table, ((0, rpad), (0, DPAD - NUM_CHANNELS)))
    bi = batch_index.astype(jnp.int32).T.reshape(-1)
    out = _make_tex_gather(batch)(table, bi)
    return out.reshape(batch, DPAD)[:, :NUM_CHANNELS]
